# baseline (speedup 1.0000x reference)
import jax
import jax.numpy as jnp
from jax import lax
from jax.experimental import pallas as pl
from jax.experimental.pallas import tpu as pltpu
from jax.experimental.pallas import tpu_sc as plsc

N_NODES = 10000
N_PAD = 10240
N_EDGES = 320000
NC, NS = 2, 16
CH = 128
CPW = 160
E_PAD = NS * CPW * CH
D_HID = 32
HF = D_HID // NC
STRIPE = N_PAD // NS
DEG_CPW = CPW // NC


def _sc_mesh():
    return plsc.VectorSubcoreMesh(core_axis_name="c", subcore_axis_name="s")


_SC_PARAMS = pltpu.CompilerParams(use_tc_tiling_on_sc=False)


def _deg_body(dst_hbm, zer_hbm, out_hbm, idx_v, ones_v, deg_sh, sem):
    c = lax.axis_index("c")
    s = lax.axis_index("s")
    pltpu.sync_copy(zer_hbm.at[pl.ds(s * STRIPE, STRIPE)],
                    deg_sh.at[pl.ds(s * STRIPE, STRIPE)])
    pltpu.async_copy(dst_hbm.at[s, pl.ds(c * DEG_CPW, DEG_CPW)], idx_v,
                     sem).wait()
    for i in range(CH // 16):
        ones_v[pl.ds(i * 16, 16)] = jnp.full((16,), 1.0, jnp.float32)
    plsc.subcore_barrier()

    def body(j, carry):
        pltpu.sync_copy(ones_v, deg_sh.at[idx_v.at[j]], add=True)
        return carry

    lax.fori_loop(0, DEG_CPW, body, 0)
    plsc.subcore_barrier()
    pltpu.sync_copy(deg_sh.at[pl.ds(s * STRIPE, STRIPE)],
                    out_hbm.at[c, pl.ds(s * STRIPE, STRIPE)])


def _make_deg_kernel():
    return pl.kernel(
        _deg_body,
        out_type=jax.ShapeDtypeStruct((NC, N_PAD), jnp.float32),
        mesh=_sc_mesh(),
        scratch_types=[
            pltpu.VMEM((DEG_CPW, CH), jnp.int32),
            pltpu.VMEM((CH,), jnp.float32),
            pltpu.VMEM_SHARED((N_PAD,), jnp.float32),
            pltpu.SemaphoreType.DMA,
        ],
        compiler_params=_SC_PARAMS,
    )


def _mega_body(h1sh_hbm, dinvrep_hbm, src_hbm, dst_hbm, zer_hbm,
               out_hbm, tab2_hbm,
               sidx_v, didx_v, rows0_v, rows1_v, accT, hT, dT, acc_sh,
               sem0, sem1):
    c = lax.axis_index("c")
    s = lax.axis_index("s")
    base = s * STRIPE
    pltpu.sync_copy(zer_hbm.at[pl.ds(base, STRIPE)],
                    acc_sh.at[pl.ds(base, STRIPE)])
    pltpu.async_copy(src_hbm.at[s], sidx_v, sem0).wait()
    pltpu.async_copy(dst_hbm.at[s], didx_v, sem0).wait()
    pltpu.sync_copy(dinvrep_hbm.at[pl.ds(base, STRIPE)], dT)
    pltpu.sync_copy(h1sh_hbm.at[c, pl.ds(base, STRIPE)], hT)
    plsc.subcore_barrier()

    def prop(tab_view):
        pltpu.async_copy(tab_view.at[sidx_v.at[0]], rows0_v, sem0)
        pltpu.async_copy(tab_view.at[sidx_v.at[1]], rows1_v, sem1)

        def body(k, carry):
            a = 2 * k
            pltpu.make_async_copy(tab_view.at[sidx_v.at[a]], rows0_v,
                                  sem0).wait()
            pltpu.sync_copy(rows0_v, acc_sh.at[didx_v.at[a]], add=True)

            @pl.when(k < CPW // 2 - 1)
            def _():
                pltpu.async_copy(tab_view.at[sidx_v.at[a + 2]], rows0_v, sem0)

            pltpu.make_async_copy(tab_view.at[sidx_v.at[a + 1]], rows1_v,
                                  sem1).wait()
            pltpu.sync_copy(rows1_v, acc_sh.at[didx_v.at[a + 1]], add=True)

            @pl.when(k < CPW // 2 - 1)
            def _():
                pltpu.async_copy(tab_view.at[sidx_v.at[a + 3]], rows1_v, sem1)

            return carry

        lax.fori_loop(0, CPW // 2, body, 0)

    prop(h1sh_hbm.at[c])
    plsc.subcore_barrier()

    pltpu.sync_copy(acc_sh.at[pl.ds(base, STRIPE)], accT)

    def mid(r, carry):
        t = (accT[r] + hT[r]) * dT[r]
        e = jnp.exp(2.0 * t)
        hT[r] = (1.0 - 2.0 / (e + 1.0)) * dT[r]
        return carry

    lax.fori_loop(0, STRIPE, mid, 0)
    pltpu.sync_copy(hT, tab2_hbm.at[c, pl.ds(base, STRIPE)])
    pltpu.sync_copy(zer_hbm.at[pl.ds(base, STRIPE)],
                    acc_sh.at[pl.ds(base, STRIPE)])
    plsc.subcore_barrier()

    prop(tab2_hbm.at[c])
    plsc.subcore_barrier()

    pltpu.sync_copy(acc_sh.at[pl.ds(base, STRIPE)], accT)

    def fin(r, carry):
        accT[r] = (accT[r] + hT[r]) * dT[r]
        return carry

    lax.fori_loop(0, STRIPE, fin, 0)
    pltpu.sync_copy(accT, out_hbm.at[c, pl.ds(base, STRIPE)])


def _make_mega_kernel():
    return pl.kernel(
        _mega_body,
        out_type=(jax.ShapeDtypeStruct((NC, N_PAD, HF), jnp.float32),
                  jax.ShapeDtypeStruct((NC, N_PAD, HF), jnp.float32)),
        mesh=_sc_mesh(),
        scratch_types=[
            pltpu.VMEM((CPW, CH), jnp.int32),
            pltpu.VMEM((CPW, CH), jnp.int32),
            pltpu.VMEM((CH, HF), jnp.float32),
            pltpu.VMEM((CH, HF), jnp.float32),
            pltpu.VMEM((STRIPE, HF), jnp.float32),
            pltpu.VMEM((STRIPE, HF), jnp.float32),
            pltpu.VMEM((STRIPE, HF), jnp.float32),
            pltpu.VMEM_SHARED((N_PAD, HF), jnp.float32),
            pltpu.SemaphoreType.DMA,
            pltpu.SemaphoreType.DMA,
        ],
        compiler_params=_SC_PARAMS,
    )


BR = 1024


def _h1s_body(x_ref, w_ref, dp_ref, o_ref, d_ref):
    deg = dp_ref[0, :] + dp_ref[1, :] + 1.0
    dinv = lax.rsqrt(deg)
    h = jnp.dot(x_ref[...], w_ref[...], preferred_element_type=jnp.float32)
    o_ref[...] = h * dinv[:, None]
    d_ref[...] = jnp.broadcast_to(dinv[:, None], (BR, HF))


def _zmat_body(p_ref, w2_ref, o_ref):
    o_ref[...] = jnp.dot(p_ref[...], w2_ref[...],
                         preferred_element_type=jnp.float32)


def _row_spec(d):
    return pl.BlockSpec((BR, d), lambda i: (i, 0))


_GRID = (N_PAD // BR,)


def _h1s_call(xp, W1, deg_part):
    return pl.pallas_call(
        _h1s_body,
        grid=_GRID,
        in_specs=[_row_spec(128),
                  pl.BlockSpec((128, D_HID), lambda i: (0, 0)),
                  pl.BlockSpec((NC, BR), lambda i: (0, i))],
        out_specs=[_row_spec(D_HID), _row_spec(HF)],
        out_shape=[jax.ShapeDtypeStruct((N_PAD, D_HID), jnp.float32),
                   jax.ShapeDtypeStruct((N_PAD, HF), jnp.float32)],
    )(xp, W1, deg_part)


def _zmat_call(p2, W2p):
    return pl.pallas_call(
        _zmat_body,
        grid=_GRID,
        in_specs=[_row_spec(D_HID),
                  pl.BlockSpec((D_HID, 128), lambda i: (0, 0))],
        out_specs=_row_spec(128),
        out_shape=jax.ShapeDtypeStruct((N_PAD, 128), jnp.float32),
    )(p2, W2p)


def kernel(x, edge_index, W1, W2):
    n = x.shape[0]
    pad_e = E_PAD - N_EDGES
    dummy = jnp.full((pad_e,), n, dtype=jnp.int32)
    srcp = jnp.concatenate([edge_index[0], dummy]).reshape(NS, CPW, CH)
    dstp = jnp.concatenate([edge_index[1], dummy]).reshape(NS, CPW, CH)
    xp = jnp.pad(x, ((0, N_PAD - n), (0, 0)))
    W2p = jnp.pad(W2, ((0, 0), (0, 128 - W2.shape[1])))
    zer1 = jnp.zeros((N_PAD,), jnp.float32)
    zer2 = jnp.zeros((N_PAD, HF), jnp.float32)

    deg_part = _make_deg_kernel()(dstp, zer1)
    h1s, dinvrep = _h1s_call(xp, W1, deg_part)
    h1sh = h1s.reshape(N_PAD, NC, HF).transpose(1, 0, 2)
    p2h, _unused_tab = _make_mega_kernel()(h1sh, dinvrep, srcp, dstp, zer2)
    p2 = p2h.transpose(1, 0, 2).reshape(N_PAD, D_HID)
    zp = _zmat_call(p2, W2p)
    z = zp[:n, :W2.shape[1]]
    return (z, z)


# speedup vs baseline: 1.0393x; 1.0393x over previous
import jax
import jax.numpy as jnp
from jax import lax
from jax.experimental import pallas as pl
from jax.experimental.pallas import tpu as pltpu
from jax.experimental.pallas import tpu_sc as plsc

N_NODES = 10000
N_PAD = 10240
N_EDGES = 320000
NC, NS = 2, 16
CH = 128
CPW = 160
E_PAD = NS * CPW * CH
D_HID = 32
HF = D_HID // NC
STRIPE = N_PAD // NS
DEG_CPW = CPW // NC


def _sc_mesh():
    return plsc.VectorSubcoreMesh(core_axis_name="c", subcore_axis_name="s")


_SC_PARAMS = pltpu.CompilerParams(use_tc_tiling_on_sc=False)


def _deg_body(dst_hbm, zer_hbm, out_hbm, idx_v, ones_v, deg_sh, sem):
    c = lax.axis_index("c")
    s = lax.axis_index("s")
    pltpu.sync_copy(zer_hbm.at[pl.ds(s * STRIPE, STRIPE)],
                    deg_sh.at[pl.ds(s * STRIPE, STRIPE)])
    pltpu.async_copy(dst_hbm.at[s, pl.ds(c * DEG_CPW, DEG_CPW)], idx_v,
                     sem).wait()
    for i in range(CH // 16):
        ones_v[pl.ds(i * 16, 16)] = jnp.full((16,), 1.0, jnp.float32)
    plsc.subcore_barrier()

    def body(j, carry):
        pltpu.sync_copy(ones_v, deg_sh.at[idx_v.at[j]], add=True)
        return carry

    lax.fori_loop(0, DEG_CPW, body, 0)
    plsc.subcore_barrier()
    pltpu.sync_copy(deg_sh.at[pl.ds(s * STRIPE, STRIPE)],
                    out_hbm.at[c, pl.ds(s * STRIPE, STRIPE)])


def _make_deg_kernel():
    return pl.kernel(
        _deg_body,
        out_type=jax.ShapeDtypeStruct((NC, N_PAD), jnp.float32),
        mesh=_sc_mesh(),
        scratch_types=[
            pltpu.VMEM((DEG_CPW, CH), jnp.int32),
            pltpu.VMEM((CH,), jnp.float32),
            pltpu.VMEM_SHARED((N_PAD,), jnp.float32),
            pltpu.SemaphoreType.DMA,
        ],
        compiler_params=_SC_PARAMS,
    )


def _mega_body(h1sh_hbm, dinvrep_hbm, src_hbm, dst_hbm, zer_hbm,
               out_hbm, tab2_hbm,
               sidx_v, didx_v, rows0_v, rows1_v, rows2_v, rows3_v,
               accT, hT, dT, acc_sh,
               gsem0, gsem1, gsem2, gsem3, ssem0, ssem1, ssem2, ssem3):
    sem0 = gsem0
    c = lax.axis_index("c")
    s = lax.axis_index("s")
    base = s * STRIPE
    pltpu.sync_copy(zer_hbm.at[pl.ds(base, STRIPE)],
                    acc_sh.at[pl.ds(base, STRIPE)])
    pltpu.async_copy(src_hbm.at[s], sidx_v, sem0).wait()
    pltpu.async_copy(dst_hbm.at[s], didx_v, sem0).wait()
    pltpu.sync_copy(dinvrep_hbm.at[pl.ds(base, STRIPE)], dT)
    pltpu.sync_copy(h1sh_hbm.at[c, pl.ds(base, STRIPE)], hT)
    plsc.subcore_barrier()

    def prop(tab_view):
        bufs = (rows0_v, rows1_v, rows2_v, rows3_v)
        gsems = (gsem0, gsem1, gsem2, gsem3)
        ssems = (ssem0, ssem1, ssem2, ssem3)

        def gather(j, slot):
            pltpu.async_copy(tab_view.at[sidx_v.at[j]], bufs[slot],
                             gsems[slot])

        def wait_gather(j, slot):
            pltpu.make_async_copy(tab_view.at[sidx_v.at[j]], bufs[slot],
                                  gsems[slot]).wait()

        def scatter(j, slot):
            pltpu.async_copy(bufs[slot], acc_sh.at[didx_v.at[j]],
                             ssems[slot], add=True)

        def wait_scatter(j, slot):
            pltpu.make_async_copy(bufs[slot], acc_sh.at[didx_v.at[j]],
                                  ssems[slot]).wait()

        gather(0, 0)
        gather(1, 1)
        for j in (0, 1):
            wait_gather(j, j)
            scatter(j, j)
            gather(j + 2, (j + 2) % 4)

        def body4(k, carry):
            base4 = 4 * k + 2
            for t in range(4):
                j = base4 + t
                slot = (2 + t) % 4
                wait_gather(j, slot)
                scatter(j, slot)
                wait_scatter(j - 2, (slot + 2) % 4)
                gather(j + 2, (slot + 2) % 4)
            return carry

        lax.fori_loop(0, (CPW - 4) // 4, body4, 0)

        for j in (CPW - 2, CPW - 1):
            wait_gather(j, j % 4)
            scatter(j, j % 4)
            wait_scatter(j - 2, (j - 2) % 4)
        wait_scatter(CPW - 2, (CPW - 2) % 4)
        wait_scatter(CPW - 1, (CPW - 1) % 4)

    prop(h1sh_hbm.at[c])
    plsc.subcore_barrier()

    pltpu.sync_copy(acc_sh.at[pl.ds(base, STRIPE)], accT)

    def mid(r, carry):
        t = (accT[r] + hT[r]) * dT[r]
        e = jnp.exp(2.0 * t)
        hT[r] = (1.0 - 2.0 / (e + 1.0)) * dT[r]
        return carry

    lax.fori_loop(0, STRIPE, mid, 0)
    pltpu.sync_copy(hT, tab2_hbm.at[c, pl.ds(base, STRIPE)])
    pltpu.sync_copy(zer_hbm.at[pl.ds(base, STRIPE)],
                    acc_sh.at[pl.ds(base, STRIPE)])
    plsc.subcore_barrier()

    prop(tab2_hbm.at[c])
    plsc.subcore_barrier()

    pltpu.sync_copy(acc_sh.at[pl.ds(base, STRIPE)], accT)

    def fin(r, carry):
        accT[r] = (accT[r] + hT[r]) * dT[r]
        return carry

    lax.fori_loop(0, STRIPE, fin, 0)
    pltpu.sync_copy(accT, out_hbm.at[c, pl.ds(base, STRIPE)])


def _make_mega_kernel():
    return pl.kernel(
        _mega_body,
        out_type=(jax.ShapeDtypeStruct((NC, N_PAD, HF), jnp.float32),
                  jax.ShapeDtypeStruct((NC, N_PAD, HF), jnp.float32)),
        mesh=_sc_mesh(),
        scratch_types=[
            pltpu.VMEM((CPW, CH), jnp.int32),
            pltpu.VMEM((CPW, CH), jnp.int32),
            pltpu.VMEM((CH, HF), jnp.float32),
            pltpu.VMEM((CH, HF), jnp.float32),
            pltpu.VMEM((CH, HF), jnp.float32),
            pltpu.VMEM((CH, HF), jnp.float32),
            pltpu.VMEM((STRIPE, HF), jnp.float32),
            pltpu.VMEM((STRIPE, HF), jnp.float32),
            pltpu.VMEM((STRIPE, HF), jnp.float32),
            pltpu.VMEM_SHARED((N_PAD, HF), jnp.float32),
        ] + [pltpu.SemaphoreType.DMA] * 8,
        compiler_params=_SC_PARAMS,
    )


BR = 1024


def _h1s_body(x_ref, w_ref, dp_ref, o_ref, d_ref):
    deg = dp_ref[0, :] + dp_ref[1, :] + 1.0
    dinv = lax.rsqrt(deg)
    h = jnp.dot(x_ref[...], w_ref[...], preferred_element_type=jnp.float32)
    o_ref[...] = h * dinv[:, None]
    d_ref[...] = jnp.broadcast_to(dinv[:, None], (BR, HF))


def _zmat_body(p_ref, w2_ref, o_ref):
    o_ref[...] = jnp.dot(p_ref[...], w2_ref[...],
                         preferred_element_type=jnp.float32)


def _row_spec(d):
    return pl.BlockSpec((BR, d), lambda i: (i, 0))


_GRID = (N_PAD // BR,)


def _h1s_call(xp, W1, deg_part):
    return pl.pallas_call(
        _h1s_body,
        grid=_GRID,
        in_specs=[_row_spec(128),
                  pl.BlockSpec((128, D_HID), lambda i: (0, 0)),
                  pl.BlockSpec((NC, BR), lambda i: (0, i))],
        out_specs=[_row_spec(D_HID), _row_spec(HF)],
        out_shape=[jax.ShapeDtypeStruct((N_PAD, D_HID), jnp.float32),
                   jax.ShapeDtypeStruct((N_PAD, HF), jnp.float32)],
    )(xp, W1, deg_part)


def _zmat_call(p2, W2p):
    return pl.pallas_call(
        _zmat_body,
        grid=_GRID,
        in_specs=[_row_spec(D_HID),
                  pl.BlockSpec((D_HID, 128), lambda i: (0, 0))],
        out_specs=_row_spec(128),
        out_shape=jax.ShapeDtypeStruct((N_PAD, 128), jnp.float32),
    )(p2, W2p)


def kernel(x, edge_index, W1, W2):
    n = x.shape[0]
    pad_e = E_PAD - N_EDGES
    dummy = jnp.full((pad_e,), n, dtype=jnp.int32)
    srcp = jnp.concatenate([edge_index[0], dummy]).reshape(NS, CPW, CH)
    dstp = jnp.concatenate([edge_index[1], dummy]).reshape(NS, CPW, CH)
    xp = jnp.pad(x, ((0, N_PAD - n), (0, 0)))
    W2p = jnp.pad(W2, ((0, 0), (0, 128 - W2.shape[1])))
    zer1 = jnp.zeros((N_PAD,), jnp.float32)
    zer2 = jnp.zeros((N_PAD, HF), jnp.float32)

    deg_part = _make_deg_kernel()(dstp, zer1)
    h1s, dinvrep = _h1s_call(xp, W1, deg_part)
    h1sh = h1s.reshape(N_PAD, NC, HF).transpose(1, 0, 2)
    p2h, _unused_tab = _make_mega_kernel()(h1sh, dinvrep, srcp, dstp, zer2)
    p2 = p2h.transpose(1, 0, 2).reshape(N_PAD, D_HID)
    zp = _zmat_call(p2, W2p)
    z = zp[:n, :W2.shape[1]]
    return (z, z)
